# baseline (speedup 1.0000x reference)
import jax
import jax.numpy as jnp
import numpy as np
from jax.experimental import pallas as pl
from jax.experimental.pallas import tpu as pltpu

N = 10000
D = 128
BM = 400
GRID = N // BM
CHUNK = 1024
NPAD = 10240
NCH = NPAD // CHUNK

f32 = jnp.float32
bf16 = jnp.bfloat16

_TILES = [(i, kc) for i in range(GRID) for kc in range(NCH)
          if kc * CHUNK + CHUNK > BM * (i + 1)]
_TI = np.array([t[0] for t in _TILES], dtype=np.int32)
_TK = np.array([t[1] for t in _TILES], dtype=np.int32)
NSTEP = len(_TILES)


def _h0_body(x_ref, w1_ref, b1_ref, out_ref):
    h0 = jnp.dot(x_ref[...], w1_ref[...], preferred_element_type=f32) + b1_ref[...]
    out_ref[...] = h0.astype(bf16)


def _pass_a_body(adj_ref, h0_ref, w2_ref, b2_ref, h2_ref, xp_ref, h2m_ref):
    i = pl.program_id(0)

    @pl.when(i == 0)
    def _zero():
        h2m_ref[...] = jnp.zeros((N, D), bf16)

    a = adj_ref[...].astype(bf16)
    h1 = jnp.maximum(jnp.dot(a, h0_ref[...], preferred_element_type=f32), 0.0)
    h2 = jnp.dot(h1, w2_ref[...], preferred_element_type=f32) + b2_ref[...]
    h2b = h2.astype(bf16)
    h2_ref[...] = h2b
    h2m_ref[pl.ds(i * BM, BM), :] = h2b

    xp_ref[...] = jnp.dot(a, h2m_ref[...], preferred_element_type=f32)


def _pass_c_body(ti_ref, tk_ref, adj_ref, h2_ref, xp_ref,
                 p1_ref, pb1_ref, p2_ref, pb2_ref, z_ref, xo_ref):
    s = pl.program_id(0)
    i = ti_ref[s]
    kc = tk_ref[s]
    start = BM * (i + 1)

    col = kc * CHUNK + jax.lax.broadcasted_iota(jnp.int32, (BM, CHUNK), 1)
    a = jnp.where((col >= start) & (col < N), adj_ref[...], 0.0).astype(bf16)
    contrib = jnp.dot(a, h2_ref[...], preferred_element_type=f32)

    first = kc == start // CHUNK

    @pl.when(first)
    def _init():
        xo_ref[...] = xp_ref[...] + contrib

    @pl.when(jnp.logical_not(first))
    def _acc():
        xo_ref[...] += contrib

    @pl.when(kc == NCH - 1)
    def _head():
        xo = xo_ref[...]
        t = jnp.maximum(
            jnp.dot(xo, p1_ref[...], preferred_element_type=f32) + pb1_ref[...],
            0.0)
        z_ref[...] = (
            jnp.dot(t, p2_ref[...], preferred_element_type=f32) + pb2_ref[...])


def kernel(x, Adj_, W1, b1, W2, b2, P1, pb1, P2, pb2):
    b1r = b1.reshape(1, D)
    b2r = b2.reshape(1, D)
    pb1r = pb1.reshape(1, D)
    pb2r = pb2.reshape(1, D)

    full = lambda r, c: pl.BlockSpec((r, c), lambda *_: (0, 0))

    h0 = pl.pallas_call(
        _h0_body,
        grid=(1,),
        in_specs=[full(N, D), full(D, D), full(1, D)],
        out_specs=pl.BlockSpec((N, D), lambda i: (0, 0)),
        out_shape=jax.ShapeDtypeStruct((N, D), bf16),
    )(x, W1, b1r)

    h2, xpart = pl.pallas_call(
        _pass_a_body,
        grid=(GRID,),
        in_specs=[
            pl.BlockSpec((BM, N), lambda i: (i, 0)),
            full(N, D), full(D, D), full(1, D),
        ],
        out_specs=[
            pl.BlockSpec((BM, D), lambda i: (i, 0)),
            pl.BlockSpec((BM, D), lambda i: (i, 0)),
        ],
        out_shape=[
            jax.ShapeDtypeStruct((N, D), bf16),
            jax.ShapeDtypeStruct((N, D), f32),
        ],
        scratch_shapes=[pltpu.VMEM((N, D), bf16)],
    )(Adj_, h0, W2, b2r)

    h2p = jnp.concatenate([h2, jnp.zeros((NPAD - N, D), bf16)], axis=0)

    z, x_out = pl.pallas_call(
        _pass_c_body,
        grid_spec=pltpu.PrefetchScalarGridSpec(
            num_scalar_prefetch=2,
            grid=(NSTEP,),
            in_specs=[
                pl.BlockSpec((BM, CHUNK), lambda s, ti, tk: (ti[s], tk[s])),
                pl.BlockSpec((CHUNK, D), lambda s, ti, tk: (tk[s], 0)),
                pl.BlockSpec((BM, D), lambda s, ti, tk: (ti[s], 0)),
                pl.BlockSpec((D, D), lambda s, ti, tk: (0, 0)),
                pl.BlockSpec((1, D), lambda s, ti, tk: (0, 0)),
                pl.BlockSpec((D, D), lambda s, ti, tk: (0, 0)),
                pl.BlockSpec((1, D), lambda s, ti, tk: (0, 0)),
            ],
            out_specs=[
                pl.BlockSpec((BM, D), lambda s, ti, tk: (ti[s], 0)),
                pl.BlockSpec((BM, D), lambda s, ti, tk: (ti[s], 0)),
            ],
        ),
        out_shape=[
            jax.ShapeDtypeStruct((N, D), f32),
            jax.ShapeDtypeStruct((N, D), f32),
        ],
    )(jnp.asarray(_TI), jnp.asarray(_TK), Adj_, h2p, xpart, P1, pb1r, P2, pb2r)

    return (z, x_out)


# speedup vs baseline: 1.3001x; 1.3001x over previous
import jax
import jax.numpy as jnp
import numpy as np
from jax.experimental import pallas as pl
from jax.experimental.pallas import tpu as pltpu

N = 10000
D = 128
BM = 400
GRID = N // BM
CHUNK = 1024
NPAD = 10240
NCH = NPAD // CHUNK

f32 = jnp.float32
bf16 = jnp.bfloat16

_TILES = [(i, kc) for i in range(GRID) for kc in range(NCH)
          if kc * CHUNK + CHUNK > BM * i]
_TI = np.array([t[0] for t in _TILES], dtype=np.int32)
_TK = np.array([t[1] for t in _TILES], dtype=np.int32)
NSTEP = len(_TILES)


def _h0_body(x_ref, w1_ref, b1_ref, out_ref):
    h0 = jnp.dot(x_ref[...], w1_ref[...], preferred_element_type=f32) + b1_ref[...]
    out_ref[...] = h0.astype(bf16)


def _pass_a_body(adj_ref, h0_ref, w2_ref, b2_ref, h2_ref, xp_ref, hcat_ref):
    i = pl.program_id(0)

    @pl.when(i == 0)
    def _init():
        hcat_ref[:, :D] = h0_ref[...]
        hcat_ref[:, D:] = jnp.zeros((N, D), bf16)

    a = adj_ref[...].astype(bf16)
    acc = jnp.dot(a, hcat_ref[...], preferred_element_type=f32)
    h1 = jnp.maximum(acc[:, :D], 0.0)
    h2 = jnp.dot(h1, w2_ref[...], preferred_element_type=f32) + b2_ref[...]
    h2b = h2.astype(bf16)
    h2_ref[...] = h2b
    hcat_ref[pl.ds(i * BM, BM), D:] = h2b

    xp_ref[...] = acc[:, D:]


def _pass_c_body(ti_ref, tk_ref, adj_ref, h2_ref, xp_ref,
                 p1_ref, pb1_ref, p2_ref, pb2_ref, z_ref, xo_ref):
    s = pl.program_id(0)
    i = ti_ref[s]
    kc = tk_ref[s]
    start = BM * i

    col = kc * CHUNK + jax.lax.broadcasted_iota(jnp.int32, (BM, CHUNK), 1)
    a = jnp.where((col >= start) & (col < N), adj_ref[...], 0.0).astype(bf16)
    contrib = jnp.dot(a, h2_ref[...], preferred_element_type=f32)

    first = kc == start // CHUNK

    @pl.when(first)
    def _init():
        xo_ref[...] = xp_ref[...] + contrib

    @pl.when(jnp.logical_not(first))
    def _acc():
        xo_ref[...] += contrib

    @pl.when(kc == NCH - 1)
    def _head():
        xo = xo_ref[...]
        t = jnp.maximum(
            jnp.dot(xo, p1_ref[...], preferred_element_type=f32) + pb1_ref[...],
            0.0)
        z_ref[...] = (
            jnp.dot(t, p2_ref[...], preferred_element_type=f32) + pb2_ref[...])


def kernel(x, Adj_, W1, b1, W2, b2, P1, pb1, P2, pb2):
    b1r = b1.reshape(1, D)
    b2r = b2.reshape(1, D)
    pb1r = pb1.reshape(1, D)
    pb2r = pb2.reshape(1, D)

    full = lambda r, c: pl.BlockSpec((r, c), lambda *_: (0, 0))

    h0 = pl.pallas_call(
        _h0_body,
        grid=(1,),
        in_specs=[full(N, D), full(D, D), full(1, D)],
        out_specs=pl.BlockSpec((N, D), lambda i: (0, 0)),
        out_shape=jax.ShapeDtypeStruct((N, D), bf16),
    )(x, W1, b1r)

    h2, xpart = pl.pallas_call(
        _pass_a_body,
        grid=(GRID,),
        in_specs=[
            pl.BlockSpec((BM, N), lambda i: (i, 0)),
            full(N, D), full(D, D), full(1, D),
        ],
        out_specs=[
            pl.BlockSpec((BM, D), lambda i: (i, 0)),
            pl.BlockSpec((BM, D), lambda i: (i, 0)),
        ],
        out_shape=[
            jax.ShapeDtypeStruct((N, D), bf16),
            jax.ShapeDtypeStruct((N, D), f32),
        ],
        scratch_shapes=[pltpu.VMEM((N, 2 * D), bf16)],
    )(Adj_, h0, W2, b2r)

    h2p = jnp.concatenate([h2, jnp.zeros((NPAD - N, D), bf16)], axis=0)

    z, x_out = pl.pallas_call(
        _pass_c_body,
        grid_spec=pltpu.PrefetchScalarGridSpec(
            num_scalar_prefetch=2,
            grid=(NSTEP,),
            in_specs=[
                pl.BlockSpec((BM, CHUNK), lambda s, ti, tk: (ti[s], tk[s])),
                pl.BlockSpec((CHUNK, D), lambda s, ti, tk: (tk[s], 0)),
                pl.BlockSpec((BM, D), lambda s, ti, tk: (ti[s], 0)),
                pl.BlockSpec((D, D), lambda s, ti, tk: (0, 0)),
                pl.BlockSpec((1, D), lambda s, ti, tk: (0, 0)),
                pl.BlockSpec((D, D), lambda s, ti, tk: (0, 0)),
                pl.BlockSpec((1, D), lambda s, ti, tk: (0, 0)),
            ],
            out_specs=[
                pl.BlockSpec((BM, D), lambda s, ti, tk: (ti[s], 0)),
                pl.BlockSpec((BM, D), lambda s, ti, tk: (ti[s], 0)),
            ],
        ),
        out_shape=[
            jax.ShapeDtypeStruct((N, D), f32),
            jax.ShapeDtypeStruct((N, D), f32),
        ],
    )(jnp.asarray(_TI), jnp.asarray(_TK), Adj_, h2p, xpart, P1, pb1r, P2, pb2r)

    return (z, x_out)


# speedup vs baseline: 1.6438x; 1.2644x over previous
import jax
import jax.numpy as jnp
import numpy as np
from jax.experimental import pallas as pl
from jax.experimental.pallas import tpu as pltpu

N = 10000
D = 128
BM = 400
GRID = N // BM
CH2 = 2000
SPC = CH2 // BM
NR2 = N // CH2
CHUNK = 1024
NPAD = 10240
NCH = NPAD // CHUNK

f32 = jnp.float32
bf16 = jnp.bfloat16

_TILES = [(I, kc) for I in range(NR2) for kc in range(NCH)
          if kc * CHUNK + CHUNK > CH2 * I]
_TI = np.array([t[0] for t in _TILES], dtype=np.int32)
_TK = np.array([t[1] for t in _TILES], dtype=np.int32)
NSTEP = len(_TILES)


def _h0_body(x_ref, w1_ref, b1_ref, out_ref):
    h0 = jnp.dot(x_ref[...], w1_ref[...], preferred_element_type=f32) + b1_ref[...]
    out_ref[...] = h0.astype(bf16)


def _pass_a_body(adj_ref, h0_ref, w2_ref, b2_ref, h2_ref, xp_ref,
                 hcat_ref, stage_ref):
    i = pl.program_id(0)

    @pl.when(i == 0)
    def _init():
        hcat_ref[:, :D] = h0_ref[...]
        hcat_ref[:, D:] = jnp.zeros((N, D), bf16)

    a = adj_ref[...].astype(bf16)
    acc = jnp.dot(a, hcat_ref[...], preferred_element_type=f32)
    h1 = jnp.maximum(acc[:, :D], 0.0)
    h2 = jnp.dot(h1, w2_ref[...], preferred_element_type=f32) + b2_ref[...]
    h2b = h2.astype(bf16)
    h2_ref[...] = h2b

    xp_ref[...] = acc[:, D:]

    slot = i % SPC
    stage_ref[pl.ds(slot * BM, BM), :] = h2b

    @pl.when(slot == SPC - 1)
    def _publish():
        hcat_ref[pl.ds((i // SPC) * CH2, CH2), D:] = stage_ref[...]


def _pass_c_body(ti_ref, tk_ref, adj_ref, h2_ref, xp_ref,
                 p1_ref, pb1_ref, p2_ref, pb2_ref, z_ref, xo_ref):
    s = pl.program_id(0)
    I = ti_ref[s]
    kc = tk_ref[s]
    start = CH2 * I

    off = start - kc * CHUNK
    row = jax.lax.broadcasted_iota(jnp.int32, (CHUNK, D), 0)
    h2c = jnp.where(row >= off, h2_ref[...], jnp.zeros((), bf16))

    def _dot_masked():
        col = kc * CHUNK + jax.lax.broadcasted_iota(jnp.int32, (CH2, CHUNK), 1)
        am = jnp.where(col < N, adj_ref[...], 0.0)
        return jnp.dot(am.astype(bf16), h2c, preferred_element_type=f32)

    def _dot_plain():
        return jnp.dot(adj_ref[...].astype(bf16), h2c,
                       preferred_element_type=f32)

    contrib = jax.lax.cond(kc == NCH - 1, _dot_masked, _dot_plain)

    first = kc == start // CHUNK

    @pl.when(first)
    def _init():
        xo_ref[...] = xp_ref[...] + contrib

    @pl.when(jnp.logical_not(first))
    def _acc():
        xo_ref[...] += contrib

    @pl.when(kc == NCH - 1)
    def _head():
        xo = xo_ref[...]
        t = jnp.maximum(
            jnp.dot(xo, p1_ref[...], preferred_element_type=f32)
            + pb1_ref[...], 0.0)
        z_ref[...] = (
            jnp.dot(t, p2_ref[...], preferred_element_type=f32) + pb2_ref[...])


def kernel(x, Adj_, W1, b1, W2, b2, P1, pb1, P2, pb2):
    b1r = b1.reshape(1, D)
    b2r = b2.reshape(1, D)
    pb1r = pb1.reshape(1, D)
    pb2r = pb2.reshape(1, D)

    full = lambda r, c: pl.BlockSpec((r, c), lambda *_: (0, 0))

    h0 = pl.pallas_call(
        _h0_body,
        grid=(1,),
        in_specs=[full(N, D), full(D, D), full(1, D)],
        out_specs=pl.BlockSpec((N, D), lambda i: (0, 0)),
        out_shape=jax.ShapeDtypeStruct((N, D), bf16),
    )(x, W1, b1r)

    h2, xpart = pl.pallas_call(
        _pass_a_body,
        grid=(GRID,),
        in_specs=[
            pl.BlockSpec((BM, N), lambda i: (i, 0)),
            full(N, D), full(D, D), full(1, D),
        ],
        out_specs=[
            pl.BlockSpec((BM, D), lambda i: (i, 0)),
            pl.BlockSpec((BM, D), lambda i: (i, 0)),
        ],
        out_shape=[
            jax.ShapeDtypeStruct((N, D), bf16),
            jax.ShapeDtypeStruct((N, D), f32),
        ],
        scratch_shapes=[
            pltpu.VMEM((N, 2 * D), bf16),
            pltpu.VMEM((CH2, D), bf16),
        ],
    )(Adj_, h0, W2, b2r)

    h2p = jnp.concatenate([h2, jnp.zeros((NPAD - N, D), bf16)], axis=0)

    z, x_out = pl.pallas_call(
        _pass_c_body,
        grid_spec=pltpu.PrefetchScalarGridSpec(
            num_scalar_prefetch=2,
            grid=(NSTEP,),
            in_specs=[
                pl.BlockSpec((CH2, CHUNK), lambda s, ti, tk: (ti[s], tk[s])),
                pl.BlockSpec((CHUNK, D), lambda s, ti, tk: (tk[s], 0)),
                pl.BlockSpec((CH2, D), lambda s, ti, tk: (ti[s], 0)),
                pl.BlockSpec((D, D), lambda s, ti, tk: (0, 0)),
                pl.BlockSpec((1, D), lambda s, ti, tk: (0, 0)),
                pl.BlockSpec((D, D), lambda s, ti, tk: (0, 0)),
                pl.BlockSpec((1, D), lambda s, ti, tk: (0, 0)),
            ],
            out_specs=[
                pl.BlockSpec((CH2, D), lambda s, ti, tk: (ti[s], 0)),
                pl.BlockSpec((CH2, D), lambda s, ti, tk: (ti[s], 0)),
            ],
        ),
        out_shape=[
            jax.ShapeDtypeStruct((N, D), f32),
            jax.ShapeDtypeStruct((N, D), f32),
        ],
    )(jnp.asarray(_TI), jnp.asarray(_TK), Adj_, h2p, xpart, P1, pb1r, P2, pb2r)

    return (z, x_out)
